# baseline (speedup 1.0000x reference)
import jax
import jax.numpy as jnp
from jax.experimental import pallas as pl
from jax.experimental.pallas import tpu as pltpu

D_MODEL = 2048
N_EXPERTS = 16
TOP_K = 2

CHUNK = 1024
NBUF = 4


def _router_kernel(x_hbm, w_ref, idx_ref, wgt_ref, probs_ref, buf, sem):
    i = pl.program_id(0)
    nchunk = pl.num_programs(0)

    def issue(c):
        slot = jax.lax.rem(c, NBUF)
        pltpu.make_async_copy(
            x_hbm.at[pl.ds(c * CHUNK, CHUNK), :], buf.at[slot], sem.at[slot]
        ).start()

    @pl.when(i == 0)
    def _prologue():
        for c in range(NBUF):
            issue(jnp.int32(c))

    slot = jax.lax.rem(i, NBUF)
    pltpu.make_async_copy(
        x_hbm.at[pl.ds(i * CHUNK, CHUNK), :], buf.at[slot], sem.at[slot]
    ).wait()

    x = buf[slot]
    w = w_ref[...]
    logits = jax.lax.dot_general(
        x, w, (((1,), (1,)), ((), ())), preferred_element_type=jnp.float32
    )

    m = jnp.max(logits, axis=-1, keepdims=True)
    e = jnp.exp(logits - m)
    z = jnp.sum(e, axis=-1, keepdims=True)
    probs = e / z
    probs_ref[...] = probs

    cols = jax.lax.broadcasted_iota(jnp.int32, probs.shape, 1)
    big = jnp.int32(N_EXPERTS)

    p1 = jnp.max(probs, axis=-1, keepdims=True)
    i1 = jnp.min(jnp.where(probs >= p1, cols, big), axis=-1, keepdims=True)
    masked = jnp.where(cols == i1, -jnp.inf, probs)
    p2 = jnp.max(masked, axis=-1, keepdims=True)
    i2 = jnp.min(jnp.where(masked >= p2, cols, big), axis=-1, keepdims=True)

    denom = p1 + p2 + 1e-09
    idx_ref[...] = jnp.concatenate([i1, i2], axis=-1)
    wgt_ref[...] = jnp.concatenate([p1 / denom, p2 / denom], axis=-1)

    @pl.when(i + NBUF < nchunk)
    def _lookahead():
        issue(i + NBUF)


def kernel(x, W):
    n = x.shape[0]
    grid = (n // CHUNK,)
    out_shapes = (
        jax.ShapeDtypeStruct((n, TOP_K), jnp.int32),
        jax.ShapeDtypeStruct((n, TOP_K), jnp.float32),
        jax.ShapeDtypeStruct((n, N_EXPERTS), jnp.float32),
    )
    top_idx, weights, probs = pl.pallas_call(
        _router_kernel,
        grid=grid,
        in_specs=[
            pl.BlockSpec(memory_space=pltpu.HBM),
            pl.BlockSpec((N_EXPERTS, D_MODEL), lambda i: (0, 0)),
        ],
        out_specs=(
            pl.BlockSpec((CHUNK, TOP_K), lambda i: (i, 0)),
            pl.BlockSpec((CHUNK, TOP_K), lambda i: (i, 0)),
            pl.BlockSpec((CHUNK, N_EXPERTS), lambda i: (i, 0)),
        ),
        out_shape=out_shapes,
        scratch_shapes=[
            pltpu.VMEM((NBUF, CHUNK, D_MODEL), jnp.float32),
            pltpu.SemaphoreType.DMA((NBUF,)),
        ],
        compiler_params=pltpu.CompilerParams(
            dimension_semantics=("arbitrary",),
        ),
    )(x, W)
    return (top_idx, weights, probs)


# speedup vs baseline: 1.4015x; 1.4015x over previous
import jax
import jax.numpy as jnp
from jax.experimental import pallas as pl
from jax.experimental.pallas import tpu as pltpu

D_MODEL = 2048
N_EXPERTS = 16
TOP_K = 2

CHUNK = 1024
NBUF = 4


def _router_kernel(x_hbm, w_ref, idx_ref, wgt_ref, probs_ref, buf, sem):
    i = pl.program_id(0)
    nchunk = pl.num_programs(0)

    def issue(c):
        slot = jax.lax.rem(c, NBUF)
        pltpu.make_async_copy(
            x_hbm.at[pl.ds(c * CHUNK, CHUNK), :], buf.at[slot], sem.at[slot]
        ).start()

    @pl.when(i == 0)
    def _prologue():
        for c in range(NBUF):
            issue(jnp.int32(c))

    slot = jax.lax.rem(i, NBUF)
    pltpu.make_async_copy(
        x_hbm.at[pl.ds(i * CHUNK, CHUNK), :], buf.at[slot], sem.at[slot]
    ).wait()

    x = buf[slot]
    w = w_ref[...]
    logits = jax.lax.dot_general(
        w, x, (((1,), (1,)), ((), ())), preferred_element_type=jnp.float32
    )

    m = jnp.max(logits, axis=0, keepdims=True)
    e = jnp.exp(logits - m)
    z = jnp.sum(e, axis=0, keepdims=True)
    probs = e / z
    probs_ref[...] = probs

    rows = jax.lax.broadcasted_iota(jnp.int32, probs.shape, 0)
    big = jnp.int32(N_EXPERTS)

    p1 = jnp.max(probs, axis=0, keepdims=True)
    i1 = jnp.min(jnp.where(probs >= p1, rows, big), axis=0, keepdims=True)
    masked = jnp.where(rows == i1, -jnp.inf, probs)
    p2 = jnp.max(masked, axis=0, keepdims=True)
    i2 = jnp.min(jnp.where(masked >= p2, rows, big), axis=0, keepdims=True)

    denom = p1 + p2 + 1e-09
    zero = jnp.zeros((4, CHUNK), jnp.float32)
    idx_ref[...] = jnp.concatenate(
        [i1.astype(jnp.float32), i2.astype(jnp.float32), zero, zero[:2]], axis=0
    )
    wgt_ref[...] = jnp.concatenate([p1 / denom, p2 / denom, zero, zero[:2]], axis=0)

    @pl.when(i + NBUF < nchunk)
    def _lookahead():
        issue(i + NBUF)


def kernel(x, W):
    n = x.shape[0]
    grid = (n // CHUNK,)
    out_shapes = (
        jax.ShapeDtypeStruct((8, n), jnp.float32),
        jax.ShapeDtypeStruct((8, n), jnp.float32),
        jax.ShapeDtypeStruct((N_EXPERTS, n), jnp.float32),
    )
    idx_t, wgt_t, probs_t = pl.pallas_call(
        _router_kernel,
        grid=grid,
        in_specs=[
            pl.BlockSpec(memory_space=pltpu.HBM),
            pl.BlockSpec((N_EXPERTS, D_MODEL), lambda i: (0, 0)),
        ],
        out_specs=(
            pl.BlockSpec((8, CHUNK), lambda i: (0, i)),
            pl.BlockSpec((8, CHUNK), lambda i: (0, i)),
            pl.BlockSpec((N_EXPERTS, CHUNK), lambda i: (0, i)),
        ),
        out_shape=out_shapes,
        scratch_shapes=[
            pltpu.VMEM((NBUF, CHUNK, D_MODEL), jnp.float32),
            pltpu.SemaphoreType.DMA((NBUF,)),
        ],
        compiler_params=pltpu.CompilerParams(
            dimension_semantics=("arbitrary",),
        ),
    )(x, W)
    top_idx = idx_t[:TOP_K].T.astype(jnp.int32)
    weights = wgt_t[:TOP_K].T
    probs = probs_t.T
    return (top_idx, weights, probs)


# speedup vs baseline: 1.4231x; 1.0154x over previous
import jax
import jax.numpy as jnp
from jax.experimental import pallas as pl
from jax.experimental.pallas import tpu as pltpu

D_MODEL = 2048
N_EXPERTS = 16
TOP_K = 2

CHUNK = 512
NBUF = 8


def _router_kernel(x_hbm, w_ref, idx_ref, wgt_ref, probs_ref, buf, sem):
    i = pl.program_id(0)
    nchunk = pl.num_programs(0)

    def issue(c):
        slot = jax.lax.rem(c, NBUF)
        pltpu.make_async_copy(
            x_hbm.at[pl.ds(c * CHUNK, CHUNK), :], buf.at[slot], sem.at[slot]
        ).start()

    @pl.when(i == 0)
    def _prologue():
        for c in range(NBUF):
            issue(jnp.int32(c))

    slot = jax.lax.rem(i, NBUF)
    pltpu.make_async_copy(
        x_hbm.at[pl.ds(i * CHUNK, CHUNK), :], buf.at[slot], sem.at[slot]
    ).wait()

    x = buf[slot]
    w = w_ref[...]
    logits = jax.lax.dot_general(
        w, x, (((1,), (1,)), ((), ())), preferred_element_type=jnp.float32
    )

    m = jnp.max(logits, axis=0, keepdims=True)
    e = jnp.exp(logits - m)
    z = jnp.sum(e, axis=0, keepdims=True)
    probs = e / z
    probs_ref[...] = probs

    rows = jax.lax.broadcasted_iota(jnp.int32, probs.shape, 0)
    big = jnp.int32(N_EXPERTS)

    p1 = jnp.max(probs, axis=0, keepdims=True)
    i1 = jnp.min(jnp.where(probs >= p1, rows, big), axis=0, keepdims=True)
    masked = jnp.where(rows == i1, -jnp.inf, probs)
    p2 = jnp.max(masked, axis=0, keepdims=True)
    i2 = jnp.min(jnp.where(masked >= p2, rows, big), axis=0, keepdims=True)

    denom = p1 + p2 + 1e-09
    zero = jnp.zeros((4, CHUNK), jnp.float32)
    idx_ref[...] = jnp.concatenate(
        [i1.astype(jnp.float32), i2.astype(jnp.float32), zero, zero[:2]], axis=0
    )
    wgt_ref[...] = jnp.concatenate([p1 / denom, p2 / denom, zero, zero[:2]], axis=0)

    @pl.when(i + NBUF < nchunk)
    def _lookahead():
        issue(i + NBUF)


def kernel(x, W):
    n = x.shape[0]
    grid = (n // CHUNK,)
    out_shapes = (
        jax.ShapeDtypeStruct((8, n), jnp.float32),
        jax.ShapeDtypeStruct((8, n), jnp.float32),
        jax.ShapeDtypeStruct((N_EXPERTS, n), jnp.float32),
    )
    idx_t, wgt_t, probs_t = pl.pallas_call(
        _router_kernel,
        grid=grid,
        in_specs=[
            pl.BlockSpec(memory_space=pltpu.HBM),
            pl.BlockSpec((N_EXPERTS, D_MODEL), lambda i: (0, 0)),
        ],
        out_specs=(
            pl.BlockSpec((8, CHUNK), lambda i: (0, i)),
            pl.BlockSpec((8, CHUNK), lambda i: (0, i)),
            pl.BlockSpec((N_EXPERTS, CHUNK), lambda i: (0, i)),
        ),
        out_shape=out_shapes,
        scratch_shapes=[
            pltpu.VMEM((NBUF, CHUNK, D_MODEL), jnp.float32),
            pltpu.SemaphoreType.DMA((NBUF,)),
        ],
        compiler_params=pltpu.CompilerParams(
            dimension_semantics=("arbitrary",),
        ),
    )(x, W)
    top_idx = idx_t[:TOP_K].T.astype(jnp.int32)
    weights = wgt_t[:TOP_K].T
    probs = probs_t.T
    return (top_idx, weights, probs)


# speedup vs baseline: 1.4238x; 1.0005x over previous
import jax
import jax.numpy as jnp
from jax.experimental import pallas as pl
from jax.experimental.pallas import tpu as pltpu

D_MODEL = 2048
N_EXPERTS = 16
TOP_K = 2

CHUNK = 256
NBUF = 16


def _router_kernel(x_hbm, w_ref, idx_ref, wgt_ref, probs_ref, buf, sem):
    i = pl.program_id(0)
    nchunk = pl.num_programs(0)

    def issue(c):
        slot = jax.lax.rem(c, NBUF)
        pltpu.make_async_copy(
            x_hbm.at[pl.ds(c * CHUNK, CHUNK), :], buf.at[slot], sem.at[slot]
        ).start()

    @pl.when(i == 0)
    def _prologue():
        for c in range(NBUF):
            issue(jnp.int32(c))

    slot = jax.lax.rem(i, NBUF)
    pltpu.make_async_copy(
        x_hbm.at[pl.ds(i * CHUNK, CHUNK), :], buf.at[slot], sem.at[slot]
    ).wait()

    x = buf[slot]
    w = w_ref[...]
    logits = jax.lax.dot_general(
        w, x, (((1,), (1,)), ((), ())), preferred_element_type=jnp.float32
    )

    m = jnp.max(logits, axis=0, keepdims=True)
    e = jnp.exp(logits - m)
    z = jnp.sum(e, axis=0, keepdims=True)
    probs = e / z
    probs_ref[...] = probs

    rows = jax.lax.broadcasted_iota(jnp.int32, probs.shape, 0)
    big = jnp.int32(N_EXPERTS)

    p1 = jnp.max(probs, axis=0, keepdims=True)
    i1 = jnp.min(jnp.where(probs >= p1, rows, big), axis=0, keepdims=True)
    masked = jnp.where(rows == i1, -jnp.inf, probs)
    p2 = jnp.max(masked, axis=0, keepdims=True)
    i2 = jnp.min(jnp.where(masked >= p2, rows, big), axis=0, keepdims=True)

    denom = p1 + p2 + 1e-09
    zero = jnp.zeros((4, CHUNK), jnp.float32)
    idx_ref[...] = jnp.concatenate(
        [i1.astype(jnp.float32), i2.astype(jnp.float32), zero, zero[:2]], axis=0
    )
    wgt_ref[...] = jnp.concatenate([p1 / denom, p2 / denom, zero, zero[:2]], axis=0)

    @pl.when(i + NBUF < nchunk)
    def _lookahead():
        issue(i + NBUF)


def kernel(x, W):
    n = x.shape[0]
    grid = (n // CHUNK,)
    out_shapes = (
        jax.ShapeDtypeStruct((8, n), jnp.float32),
        jax.ShapeDtypeStruct((8, n), jnp.float32),
        jax.ShapeDtypeStruct((N_EXPERTS, n), jnp.float32),
    )
    idx_t, wgt_t, probs_t = pl.pallas_call(
        _router_kernel,
        grid=grid,
        in_specs=[
            pl.BlockSpec(memory_space=pltpu.HBM),
            pl.BlockSpec((N_EXPERTS, D_MODEL), lambda i: (0, 0)),
        ],
        out_specs=(
            pl.BlockSpec((8, CHUNK), lambda i: (0, i)),
            pl.BlockSpec((8, CHUNK), lambda i: (0, i)),
            pl.BlockSpec((N_EXPERTS, CHUNK), lambda i: (0, i)),
        ),
        out_shape=out_shapes,
        scratch_shapes=[
            pltpu.VMEM((NBUF, CHUNK, D_MODEL), jnp.float32),
            pltpu.SemaphoreType.DMA((NBUF,)),
        ],
        compiler_params=pltpu.CompilerParams(
            dimension_semantics=("arbitrary",),
        ),
    )(x, W)
    top_idx = idx_t[:TOP_K].T.astype(jnp.int32)
    weights = wgt_t[:TOP_K].T
    probs = probs_t.T
    return (top_idx, weights, probs)
